# Initial kernel scaffold; baseline (speedup 1.0000x reference)
import jax
import jax.numpy as jnp
from jax.experimental import pallas as pl


def kernel(token, embeddings):
    raise NotImplementedError("write your pallas kernel here")


# speedup vs baseline: 4.8083x; 4.8083x over previous
import functools

import jax
import jax.numpy as jnp
from jax import lax
from jax.experimental import pallas as pl
from jax.experimental.pallas import tpu as pltpu
from jax.experimental.pallas import tpu_sc as plsc

NC = 2
NS = 16
NW = NC * NS
CHUNK = 1024


@functools.lru_cache(maxsize=None)
def _build(n_rows: int, d: int):
  assert n_rows % (NW * CHUNK) == 0
  b_per_w = n_rows // NW
  nchunk = b_per_w // CHUNK

  mesh = plsc.VectorSubcoreMesh(
      core_axis_name="c", subcore_axis_name="s", num_cores=NC, num_subcores=NS
  )

  @functools.partial(
      pl.kernel,
      mesh=mesh,
      out_type=jax.ShapeDtypeStruct((n_rows, d), jnp.float32),
      scratch_types=[
          pltpu.VMEM((CHUNK,), jnp.int32),
          pltpu.VMEM((CHUNK, d), jnp.float32),
          pltpu.SemaphoreType.DMA,
      ],
      compiler_params=pltpu.CompilerParams(use_tc_tiling_on_sc=False),
  )
  def gather(idx_hbm, table_hbm, out_hbm, idx_v, rows_v, gsem):
    wid = lax.axis_index("s") * NC + lax.axis_index("c")
    base = wid * b_per_w

    @pl.loop(0, nchunk)
    def _chunk(g):
      off = base + g * CHUNK
      pltpu.sync_copy(idx_hbm.at[pl.ds(off, CHUNK)], idx_v)
      pltpu.async_copy(table_hbm.at[idx_v], rows_v, gsem).wait()
      pltpu.sync_copy(rows_v, out_hbm.at[pl.ds(off, CHUNK)])

  return gather


def kernel(token, embeddings):
  b, s = token.shape
  n = b * s
  d = embeddings.shape[1]
  idx = token.reshape(n).astype(jnp.int32)
  out = _build(n, d)(idx, embeddings)
  return out.reshape(b, s, d)


# speedup vs baseline: 5.0275x; 1.0456x over previous
import functools

import jax
import jax.numpy as jnp
from jax import lax
from jax.experimental import pallas as pl
from jax.experimental.pallas import tpu as pltpu
from jax.experimental.pallas import tpu_sc as plsc

NC = 2
NS = 16
NW = NC * NS
CHUNK = 512
NBUF = 4
LAG = 2


@functools.lru_cache(maxsize=None)
def _build(n_rows: int, d: int):
  assert n_rows % (NW * CHUNK * NBUF) == 0
  b_per_w = n_rows // NW
  nchunk = b_per_w // CHUNK

  mesh = plsc.VectorSubcoreMesh(
      core_axis_name="c", subcore_axis_name="s", num_cores=NC, num_subcores=NS
  )

  @functools.partial(
      pl.kernel,
      mesh=mesh,
      out_type=jax.ShapeDtypeStruct((n_rows, d), jnp.float32),
      scratch_types=[
          pltpu.VMEM((NBUF, CHUNK), jnp.int32),
          pltpu.VMEM((NBUF, CHUNK, d), jnp.float32),
          [pltpu.SemaphoreType.DMA] * NBUF,
          [pltpu.SemaphoreType.DMA] * NBUF,
      ],
      compiler_params=pltpu.CompilerParams(use_tc_tiling_on_sc=False),
  )
  def gather(idx_hbm, table_hbm, out_hbm, idx_v, rows_v, gsem, osem):
    wid = lax.axis_index("s") * NC + lax.axis_index("c")
    base = wid * b_per_w

    def front(g, b, wait_out):
      if wait_out:
        pltpu.make_async_copy(
            rows_v.at[b], out_hbm.at[pl.ds(0, CHUNK)], osem[b]
        ).wait()
      pltpu.sync_copy(idx_hbm.at[pl.ds(base + g * CHUNK, CHUNK)], idx_v.at[b])
      pltpu.async_copy(table_hbm.at[idx_v.at[b]], rows_v.at[b], gsem[b])

    def back(g, b):
      pltpu.make_async_copy(
          table_hbm.at[idx_v.at[b]], rows_v.at[b], gsem[b]
      ).wait()
      pltpu.async_copy(rows_v.at[b], out_hbm.at[pl.ds(base + g * CHUNK, CHUNK)], osem[b])

    for g in range(NBUF):
      front(g, g, wait_out=False)
    for g in range(NBUF - LAG):
      back(g, g)

    @pl.loop(1, nchunk // NBUF)
    def _blk(i):
      for b in range(NBUF):
        g = i * NBUF + b
        back(g - LAG, (b - LAG) % NBUF)
        front(g, b, wait_out=True)

    for k in range(LAG):
      g = nchunk - LAG + k
      back(g, g % NBUF)
    for b in range(NBUF):
      pltpu.make_async_copy(
          rows_v.at[b], out_hbm.at[pl.ds(0, CHUNK)], osem[b]
      ).wait()

  return gather


def kernel(token, embeddings):
  b, s = token.shape
  n = b * s
  d = embeddings.shape[1]
  idx = token.reshape(n).astype(jnp.int32)
  out = _build(n, d)(idx, embeddings)
  return out.reshape(b, s, d)
